# baseline (speedup 1.0000x reference)
import jax
import jax.numpy as jnp
from jax import lax
from jax.experimental import pallas as pl
from jax.experimental.pallas import tpu as pltpu

L = 2048
D_MODEL = 768
N_HEADS = 12
HEAD_DIM = 64
HALF = 32
BLOCK = 32
N_BLOCKS = L // BLOCK
N_SEL = 4
SEL_TOK = N_SEL * BLOCK
N_IDX_HEADS = 4
IDX_HD = D_MODEL // N_IDX_HEADS
ROWS = 256

_BF = jnp.bfloat16


def _proj_kernel(x_ref, wq_ref, wk_ref, wv_ref, cos_ref, sin_ref,
                 q_ref, k_ref, v_ref, xb_ref):
    x = x_ref[...]
    x16 = x.astype(_BF)
    cos = cos_ref[...]
    sin = sin_ref[...]
    lane = lax.broadcasted_iota(jnp.int32, (ROWS, D_MODEL), 1)
    first_half = (lane % HEAD_DIM) < HALF

    def rope_proj(w_ref, o_ref):
        t = lax.dot_general(x16, w_ref[...], (((1,), (0,)), ((), ())),
                            preferred_element_type=jnp.float32).astype(_BF)
        swap = jnp.where(first_half,
                         jnp.roll(t, -HALF, axis=1),
                         jnp.roll(t, HALF, axis=1))
        o_ref[...] = t * cos + swap * sin

    rope_proj(wq_ref, q_ref)
    rope_proj(wk_ref, k_ref)
    v = lax.dot_general(x16, wv_ref[...], (((1,), (0,)), ((), ())),
                        preferred_element_type=jnp.float32)
    v_ref[...] = v.astype(_BF)
    xb_ref[...] = jnp.mean(x.reshape(ROWS // BLOCK, BLOCK, D_MODEL), axis=1)


def _indexer_kernel(xb_ref, wqi_ref, wki_ref, wgi_ref, starts_ref):
    xb = xb_ref[...]
    q = lax.dot_general(xb, wqi_ref[...], (((1,), (0,)), ((), ())),
                        preferred_element_type=jnp.float32)
    k = lax.dot_general(xb, wki_ref[...], (((1,), (0,)), ((), ())),
                        preferred_element_type=jnp.float32)
    w = lax.dot_general(xb, wgi_ref[...], (((1,), (0,)), ((), ())),
                        preferred_element_type=jnp.float32)
    scores = jnp.zeros((N_BLOCKS, N_BLOCKS), jnp.float32)
    for h in range(N_IDX_HEADS):
        qh = q[:, h * IDX_HD:(h + 1) * IDX_HD]
        kh = k[:, h * IDX_HD:(h + 1) * IDX_HD]
        qk = lax.dot_general(qh, kh, (((1,), (1,)), ((), ())),
                             preferred_element_type=jnp.float32)
        scores = scores + jnp.maximum(qk, 0.0) * w[:, h:h + 1]
    iota = lax.broadcasted_iota(jnp.int32, (N_BLOCKS, N_BLOCKS), 1)
    cols = []
    for _ in range(N_SEL):
        m = jnp.max(scores, axis=1, keepdims=True)
        idx = jnp.min(jnp.where(scores == m, iota, N_BLOCKS), axis=1,
                      keepdims=True)
        cols.append(idx)
        scores = jnp.where(iota == idx, -jnp.inf, scores)
    starts_ref[...] = jnp.concatenate(cols, axis=1) * BLOCK


QB_PER_STEP = 32


def _attn_kernel(starts_ref, q_ref, k_ref, v_ref, wo_ref, o_ref,
                 ones_ref, acc_ref):
    step = pl.program_id(0)

    @pl.when(step == 0)
    def _():
        ones_ref[...] = jnp.ones((SEL_TOK, HEAD_DIM), _BF)

    ones = ones_ref[...]
    for qq in range(QB_PER_STEP):
        qb = step * QB_PER_STEP + qq
        ksel = jnp.concatenate(
            [k_ref[pl.ds(pl.multiple_of(starts_ref[qb * N_SEL + s], BLOCK),
                         BLOCK), :] for s in range(N_SEL)], axis=0)
        vsel = jnp.concatenate(
            [v_ref[pl.ds(pl.multiple_of(starts_ref[qb * N_SEL + s], BLOCK),
                         BLOCK), :] for s in range(N_SEL)], axis=0)
        q = q_ref[qq * BLOCK:(qq + 1) * BLOCK, :]
        es = []
        for h in range(N_HEADS):
            sl = slice(h * HEAD_DIM, (h + 1) * HEAD_DIM)
            s32 = lax.dot_general(q[:, sl], ksel[:, sl],
                                  (((1,), (1,)), ((), ())),
                                  preferred_element_type=jnp.float32)
            sc = s32.astype(_BF).astype(jnp.float32) * 0.125
            es.append(jnp.exp(jnp.minimum(sc, 80.0)).astype(_BF))
        pvs, dens = [], []
        for h in range(N_HEADS):
            sl = slice(h * HEAD_DIM, (h + 1) * HEAD_DIM)
            dens.append(lax.dot_general(es[h], ones, (((1,), (0,)), ((), ())),
                                        preferred_element_type=jnp.float32))
            pvs.append(lax.dot_general(es[h], vsel[:, sl],
                                       (((1,), (0,)), ((), ())),
                                       preferred_element_type=jnp.float32))
        for h in range(N_HEADS):
            sl = slice(h * HEAD_DIM, (h + 1) * HEAD_DIM)
            acc_ref[qq * BLOCK:(qq + 1) * BLOCK, sl] = (
                pvs[h] / dens[h]).astype(_BF)
    o_ref[...] = lax.dot_general(acc_ref[...], wo_ref[...],
                                 (((1,), (0,)), ((), ())),
                                 preferred_element_type=jnp.float32).astype(_BF)


def _full(shape):
    nd = len(shape)
    return pl.BlockSpec(shape, lambda *_: (0,) * nd)


def kernel(x, wq, wk, wv, wo, idx_wq, idx_wk, idx_wg):
    x2 = x[0]
    wq16 = wq.astype(_BF)
    wk16 = wk.astype(_BF)
    wv16 = wv.astype(_BF)
    wo16 = wo.astype(_BF)

    inv = 1.0 / (10000.0 ** (jnp.arange(HALF, dtype=jnp.float32) / HALF))
    fr = jnp.outer(jnp.arange(L, dtype=jnp.float32), inv)
    c, s = jnp.cos(fr), jnp.sin(fr)
    cos = jnp.tile(jnp.concatenate([c, c], axis=1), (1, N_HEADS)).astype(_BF)
    sin = jnp.tile(jnp.concatenate([-s, s], axis=1), (1, N_HEADS)).astype(_BF)

    q_r, k_r, v, xb = pl.pallas_call(
        _proj_kernel,
        grid=(L // ROWS,),
        in_specs=[
            pl.BlockSpec((ROWS, D_MODEL), lambda i: (i, 0)),
            _full((D_MODEL, D_MODEL)),
            _full((D_MODEL, D_MODEL)),
            _full((D_MODEL, D_MODEL)),
            pl.BlockSpec((ROWS, D_MODEL), lambda i: (i, 0)),
            pl.BlockSpec((ROWS, D_MODEL), lambda i: (i, 0)),
        ],
        out_specs=[
            pl.BlockSpec((ROWS, D_MODEL), lambda i: (i, 0)),
            pl.BlockSpec((ROWS, D_MODEL), lambda i: (i, 0)),
            pl.BlockSpec((ROWS, D_MODEL), lambda i: (i, 0)),
            pl.BlockSpec((ROWS // BLOCK, D_MODEL), lambda i: (i, 0)),
        ],
        out_shape=[
            jax.ShapeDtypeStruct((L, D_MODEL), _BF),
            jax.ShapeDtypeStruct((L, D_MODEL), _BF),
            jax.ShapeDtypeStruct((L, D_MODEL), _BF),
            jax.ShapeDtypeStruct((N_BLOCKS, D_MODEL), jnp.float32),
        ],
    )(x2, wq16, wk16, wv16, cos, sin)

    starts = pl.pallas_call(
        _indexer_kernel,
        in_specs=[
            _full((N_BLOCKS, D_MODEL)),
            _full((D_MODEL, D_MODEL)),
            _full((D_MODEL, D_MODEL)),
            _full((D_MODEL, N_IDX_HEADS)),
        ],
        out_specs=_full((N_BLOCKS, N_SEL)),
        out_shape=jax.ShapeDtypeStruct((N_BLOCKS, N_SEL), jnp.int32),
    )(xb, idx_wq, idx_wk, idx_wg)

    out = pl.pallas_call(
        _attn_kernel,
        grid_spec=pltpu.PrefetchScalarGridSpec(
            num_scalar_prefetch=1,
            grid=(N_BLOCKS // QB_PER_STEP,),
            in_specs=[
                pl.BlockSpec((QB_PER_STEP * BLOCK, D_MODEL),
                             lambda i, sref: (i, 0)),
                pl.BlockSpec((L, D_MODEL), lambda i, sref: (0, 0)),
                pl.BlockSpec((L, D_MODEL), lambda i, sref: (0, 0)),
                pl.BlockSpec((D_MODEL, D_MODEL), lambda i, sref: (0, 0)),
            ],
            out_specs=pl.BlockSpec((QB_PER_STEP * BLOCK, D_MODEL),
                                   lambda i, sref: (i, 0)),
            scratch_shapes=[
                pltpu.VMEM((SEL_TOK, HEAD_DIM), _BF),
                pltpu.VMEM((QB_PER_STEP * BLOCK, D_MODEL), _BF),
            ],
        ),
        out_shape=jax.ShapeDtypeStruct((L, D_MODEL), _BF),
    )(starts.reshape(-1), q_r, k_r, v, wo16)

    return out[None]


# speedup vs baseline: 1.0787x; 1.0787x over previous
import jax
import jax.numpy as jnp
from jax import lax
from jax.experimental import pallas as pl
from jax.experimental.pallas import tpu as pltpu

L = 2048
D_MODEL = 768
N_HEADS = 12
HEAD_DIM = 64
HALF = 32
BLOCK = 32
N_BLOCKS = L // BLOCK
N_SEL = 4
SEL_TOK = N_SEL * BLOCK
N_IDX_HEADS = 4
IDX_HD = D_MODEL // N_IDX_HEADS
ROWS = 256

_BF = jnp.bfloat16


def _proj_kernel(x_ref, wq_ref, wk_ref, wv_ref, cos_ref, sin_ref,
                 q_ref, k_ref, v_ref, xb_ref):
    x = x_ref[...]
    x16 = x.astype(_BF)
    cos = cos_ref[...]
    sin = sin_ref[...]
    lane = lax.broadcasted_iota(jnp.int32, (ROWS, D_MODEL), 1)
    first_half = (lane % HEAD_DIM) < HALF

    def rope_proj(w_ref, o_ref):
        t = lax.dot_general(x16, w_ref[...], (((1,), (0,)), ((), ())),
                            preferred_element_type=jnp.float32).astype(_BF)
        swap = jnp.where(first_half,
                         jnp.roll(t, -HALF, axis=1),
                         jnp.roll(t, HALF, axis=1))
        o_ref[...] = t * cos + swap * sin

    rope_proj(wq_ref, q_ref)
    rope_proj(wk_ref, k_ref)
    v = lax.dot_general(x16, wv_ref[...], (((1,), (0,)), ((), ())),
                        preferred_element_type=jnp.float32)
    v_ref[...] = v.astype(_BF)
    xb_ref[...] = jnp.mean(x.reshape(ROWS // BLOCK, BLOCK, D_MODEL), axis=1)


def _indexer_kernel(xb_ref, wqi_ref, wki_ref, wgi_ref, starts_ref):
    xb = xb_ref[...]
    q = lax.dot_general(xb, wqi_ref[...], (((1,), (0,)), ((), ())),
                        preferred_element_type=jnp.float32)
    k = lax.dot_general(xb, wki_ref[...], (((1,), (0,)), ((), ())),
                        preferred_element_type=jnp.float32)
    w = lax.dot_general(xb, wgi_ref[...], (((1,), (0,)), ((), ())),
                        preferred_element_type=jnp.float32)
    scores = jnp.zeros((N_BLOCKS, N_BLOCKS), jnp.float32)
    for h in range(N_IDX_HEADS):
        qh = q[:, h * IDX_HD:(h + 1) * IDX_HD]
        kh = k[:, h * IDX_HD:(h + 1) * IDX_HD]
        qk = lax.dot_general(qh, kh, (((1,), (1,)), ((), ())),
                             preferred_element_type=jnp.float32)
        scores = scores + jnp.maximum(qk, 0.0) * w[:, h:h + 1]
    iota = lax.broadcasted_iota(jnp.int32, (N_BLOCKS, N_BLOCKS), 1)
    cols = []
    for _ in range(N_SEL):
        m = jnp.max(scores, axis=1, keepdims=True)
        idx = jnp.min(jnp.where(scores == m, iota, N_BLOCKS), axis=1,
                      keepdims=True)
        cols.append(idx)
        scores = jnp.where(iota == idx, -jnp.inf, scores)
    starts_ref[...] = jnp.concatenate(cols, axis=1) * BLOCK


QB_PER_STEP = 16


def _attn_kernel(starts_ref, q_ref, k_ref, v_ref, wo_ref, o_ref,
                 ones_ref, acc_ref):
    step = pl.program_id(0)

    @pl.when(step == 0)
    def _():
        ones_ref[...] = jnp.ones((SEL_TOK, HEAD_DIM), _BF)

    ones = ones_ref[...]
    for qq in range(QB_PER_STEP):
        qb = step * QB_PER_STEP + qq
        ksel = jnp.concatenate(
            [k_ref[pl.ds(pl.multiple_of(starts_ref[qb * N_SEL + s], BLOCK),
                         BLOCK), :] for s in range(N_SEL)], axis=0)
        vsel = jnp.concatenate(
            [v_ref[pl.ds(pl.multiple_of(starts_ref[qb * N_SEL + s], BLOCK),
                         BLOCK), :] for s in range(N_SEL)], axis=0)
        q = q_ref[qq * BLOCK:(qq + 1) * BLOCK, :]
        es = []
        for h in range(N_HEADS):
            sl = slice(h * HEAD_DIM, (h + 1) * HEAD_DIM)
            s32 = lax.dot_general(q[:, sl], ksel[:, sl],
                                  (((1,), (1,)), ((), ())),
                                  preferred_element_type=jnp.float32)
            sc = s32.astype(_BF).astype(jnp.float32) * 0.125
            es.append(jnp.exp(jnp.minimum(sc, 80.0)).astype(_BF))
        pvs, dens = [], []
        for h in range(N_HEADS):
            sl = slice(h * HEAD_DIM, (h + 1) * HEAD_DIM)
            dens.append(jnp.sum(es[h].astype(jnp.float32), axis=1,
                                keepdims=True))
            pvs.append(lax.dot_general(es[h], vsel[:, sl],
                                       (((1,), (0,)), ((), ())),
                                       preferred_element_type=jnp.float32))
        for h in range(N_HEADS):
            sl = slice(h * HEAD_DIM, (h + 1) * HEAD_DIM)
            acc_ref[qq * BLOCK:(qq + 1) * BLOCK, sl] = (
                pvs[h] / dens[h]).astype(_BF)
    o_ref[...] = lax.dot_general(acc_ref[...], wo_ref[...],
                                 (((1,), (0,)), ((), ())),
                                 preferred_element_type=jnp.float32).astype(_BF)


def _full(shape):
    nd = len(shape)
    return pl.BlockSpec(shape, lambda *_: (0,) * nd)


def kernel(x, wq, wk, wv, wo, idx_wq, idx_wk, idx_wg):
    x2 = x[0]
    wq16 = wq.astype(_BF)
    wk16 = wk.astype(_BF)
    wv16 = wv.astype(_BF)
    wo16 = wo.astype(_BF)

    inv = 1.0 / (10000.0 ** (jnp.arange(HALF, dtype=jnp.float32) / HALF))
    fr = jnp.outer(jnp.arange(L, dtype=jnp.float32), inv)
    c, s = jnp.cos(fr), jnp.sin(fr)
    cos = jnp.tile(jnp.concatenate([c, c], axis=1), (1, N_HEADS)).astype(_BF)
    sin = jnp.tile(jnp.concatenate([-s, s], axis=1), (1, N_HEADS)).astype(_BF)

    q_r, k_r, v, xb = pl.pallas_call(
        _proj_kernel,
        grid=(L // ROWS,),
        in_specs=[
            pl.BlockSpec((ROWS, D_MODEL), lambda i: (i, 0)),
            _full((D_MODEL, D_MODEL)),
            _full((D_MODEL, D_MODEL)),
            _full((D_MODEL, D_MODEL)),
            pl.BlockSpec((ROWS, D_MODEL), lambda i: (i, 0)),
            pl.BlockSpec((ROWS, D_MODEL), lambda i: (i, 0)),
        ],
        out_specs=[
            pl.BlockSpec((ROWS, D_MODEL), lambda i: (i, 0)),
            pl.BlockSpec((ROWS, D_MODEL), lambda i: (i, 0)),
            pl.BlockSpec((ROWS, D_MODEL), lambda i: (i, 0)),
            pl.BlockSpec((ROWS // BLOCK, D_MODEL), lambda i: (i, 0)),
        ],
        out_shape=[
            jax.ShapeDtypeStruct((L, D_MODEL), _BF),
            jax.ShapeDtypeStruct((L, D_MODEL), _BF),
            jax.ShapeDtypeStruct((L, D_MODEL), _BF),
            jax.ShapeDtypeStruct((N_BLOCKS, D_MODEL), jnp.float32),
        ],
    )(x2, wq16, wk16, wv16, cos, sin)

    starts = pl.pallas_call(
        _indexer_kernel,
        in_specs=[
            _full((N_BLOCKS, D_MODEL)),
            _full((D_MODEL, D_MODEL)),
            _full((D_MODEL, D_MODEL)),
            _full((D_MODEL, N_IDX_HEADS)),
        ],
        out_specs=_full((N_BLOCKS, N_SEL)),
        out_shape=jax.ShapeDtypeStruct((N_BLOCKS, N_SEL), jnp.int32),
    )(xb, idx_wq, idx_wk, idx_wg)

    out = pl.pallas_call(
        _attn_kernel,
        grid_spec=pltpu.PrefetchScalarGridSpec(
            num_scalar_prefetch=1,
            grid=(N_BLOCKS // QB_PER_STEP,),
            in_specs=[
                pl.BlockSpec((QB_PER_STEP * BLOCK, D_MODEL),
                             lambda i, sref: (i, 0)),
                pl.BlockSpec((L, D_MODEL), lambda i, sref: (0, 0)),
                pl.BlockSpec((L, D_MODEL), lambda i, sref: (0, 0)),
                pl.BlockSpec((D_MODEL, D_MODEL), lambda i, sref: (0, 0)),
            ],
            out_specs=pl.BlockSpec((QB_PER_STEP * BLOCK, D_MODEL),
                                   lambda i, sref: (i, 0)),
            scratch_shapes=[
                pltpu.VMEM((SEL_TOK, HEAD_DIM), _BF),
                pltpu.VMEM((QB_PER_STEP * BLOCK, D_MODEL), _BF),
            ],
        ),
        out_shape=jax.ShapeDtypeStruct((L, D_MODEL), _BF),
    )(starts.reshape(-1), q_r, k_r, v, wo16)

    return out[None]


# speedup vs baseline: 1.0789x; 1.0002x over previous
import jax
import jax.numpy as jnp
from jax import lax
from jax.experimental import pallas as pl
from jax.experimental.pallas import tpu as pltpu

L = 2048
D_MODEL = 768
N_HEADS = 12
HEAD_DIM = 64
HALF = 32
BLOCK = 32
N_BLOCKS = L // BLOCK
N_SEL = 4
SEL_TOK = N_SEL * BLOCK
N_IDX_HEADS = 4
IDX_HD = D_MODEL // N_IDX_HEADS
ROWS = 256

_BF = jnp.bfloat16


def _proj_kernel(x_ref, wq_ref, wk_ref, wv_ref, cos_ref, sin_ref,
                 q_ref, k_ref, v_ref, xb_ref):
    x = x_ref[...]
    x16 = x.astype(_BF)
    cos = cos_ref[...]
    sin = sin_ref[...]
    lane = lax.broadcasted_iota(jnp.int32, (ROWS, D_MODEL), 1)
    first_half = (lane % HEAD_DIM) < HALF

    def rope_proj(w_ref, o_ref):
        t = lax.dot_general(x16, w_ref[...], (((1,), (0,)), ((), ())),
                            preferred_element_type=jnp.float32).astype(_BF)
        swap = jnp.where(first_half,
                         jnp.roll(t, -HALF, axis=1),
                         jnp.roll(t, HALF, axis=1))
        o_ref[...] = t * cos + swap * sin

    rope_proj(wq_ref, q_ref)
    rope_proj(wk_ref, k_ref)
    v = lax.dot_general(x16, wv_ref[...], (((1,), (0,)), ((), ())),
                        preferred_element_type=jnp.float32)
    v_ref[...] = v.astype(_BF)
    xb_ref[...] = jnp.mean(x.reshape(ROWS // BLOCK, BLOCK, D_MODEL), axis=1)


def _indexer_kernel(xb_ref, wqi_ref, wki_ref, wgi_ref, starts_ref):
    xb = xb_ref[...]
    q = lax.dot_general(xb, wqi_ref[...], (((1,), (0,)), ((), ())),
                        preferred_element_type=jnp.float32)
    k = lax.dot_general(xb, wki_ref[...], (((1,), (0,)), ((), ())),
                        preferred_element_type=jnp.float32)
    w = lax.dot_general(xb, wgi_ref[...], (((1,), (0,)), ((), ())),
                        preferred_element_type=jnp.float32)
    scores = jnp.zeros((N_BLOCKS, N_BLOCKS), jnp.float32)
    for h in range(N_IDX_HEADS):
        qh = q[:, h * IDX_HD:(h + 1) * IDX_HD]
        kh = k[:, h * IDX_HD:(h + 1) * IDX_HD]
        qk = lax.dot_general(qh, kh, (((1,), (1,)), ((), ())),
                             preferred_element_type=jnp.float32)
        scores = scores + jnp.maximum(qk, 0.0) * w[:, h:h + 1]
    iota = lax.broadcasted_iota(jnp.int32, (N_BLOCKS, N_BLOCKS), 1)
    cols = []
    for _ in range(N_SEL):
        m = jnp.max(scores, axis=1, keepdims=True)
        idx = jnp.min(jnp.where(scores == m, iota, N_BLOCKS), axis=1,
                      keepdims=True)
        cols.append(idx)
        scores = jnp.where(iota == idx, -jnp.inf, scores)
    starts_ref[...] = jnp.concatenate(cols, axis=1) * BLOCK


QB_PER_STEP = 16


def _attn_kernel(starts_ref, q_ref, k_ref, v_ref, wo_ref, o_ref, acc_ref):
    step = pl.program_id(0)
    for qq in range(QB_PER_STEP):
        qb = step * QB_PER_STEP + qq
        ksel = jnp.concatenate(
            [k_ref[pl.ds(pl.multiple_of(starts_ref[qb * N_SEL + s], BLOCK),
                         BLOCK), :] for s in range(N_SEL)], axis=0)
        vsel = jnp.concatenate(
            [v_ref[pl.ds(pl.multiple_of(starts_ref[qb * N_SEL + s], BLOCK),
                         BLOCK), :] for s in range(N_SEL)], axis=0)
        q = q_ref[qq * BLOCK:(qq + 1) * BLOCK, :]
        es = []
        for h in range(N_HEADS):
            sl = slice(h * HEAD_DIM, (h + 1) * HEAD_DIM)
            s32 = lax.dot_general(q[:, sl], ksel[:, sl],
                                  (((1,), (1,)), ((), ())),
                                  preferred_element_type=jnp.float32)
            sc = s32.astype(_BF).astype(jnp.float32) * 0.125
            es.append(jnp.exp(jnp.minimum(sc, 80.0)).astype(_BF))
        pvs, dens = [], []
        for h in range(N_HEADS):
            sl = slice(h * HEAD_DIM, (h + 1) * HEAD_DIM)
            dens.append(jnp.sum(es[h].astype(jnp.float32), axis=1,
                                keepdims=True))
            pvs.append(lax.dot_general(es[h], vsel[:, sl],
                                       (((1,), (0,)), ((), ())),
                                       preferred_element_type=jnp.float32))
        for h in range(N_HEADS):
            sl = slice(h * HEAD_DIM, (h + 1) * HEAD_DIM)
            acc_ref[qq * BLOCK:(qq + 1) * BLOCK, sl] = (
                pvs[h] / dens[h]).astype(_BF)
    o_ref[...] = lax.dot_general(acc_ref[...], wo_ref[...],
                                 (((1,), (0,)), ((), ())),
                                 preferred_element_type=jnp.float32).astype(_BF)


def _full(shape):
    nd = len(shape)
    return pl.BlockSpec(shape, lambda *_: (0,) * nd)


def kernel(x, wq, wk, wv, wo, idx_wq, idx_wk, idx_wg):
    x2 = x[0]
    wq16 = wq.astype(_BF)
    wk16 = wk.astype(_BF)
    wv16 = wv.astype(_BF)
    wo16 = wo.astype(_BF)

    inv = 1.0 / (10000.0 ** (jnp.arange(HALF, dtype=jnp.float32) / HALF))
    fr = jnp.outer(jnp.arange(L, dtype=jnp.float32), inv)
    c, s = jnp.cos(fr), jnp.sin(fr)
    cos = jnp.tile(jnp.concatenate([c, c], axis=1), (1, N_HEADS)).astype(_BF)
    sin = jnp.tile(jnp.concatenate([-s, s], axis=1), (1, N_HEADS)).astype(_BF)

    q_r, k_r, v, xb = pl.pallas_call(
        _proj_kernel,
        grid=(L // ROWS,),
        in_specs=[
            pl.BlockSpec((ROWS, D_MODEL), lambda i: (i, 0)),
            _full((D_MODEL, D_MODEL)),
            _full((D_MODEL, D_MODEL)),
            _full((D_MODEL, D_MODEL)),
            pl.BlockSpec((ROWS, D_MODEL), lambda i: (i, 0)),
            pl.BlockSpec((ROWS, D_MODEL), lambda i: (i, 0)),
        ],
        out_specs=[
            pl.BlockSpec((ROWS, D_MODEL), lambda i: (i, 0)),
            pl.BlockSpec((ROWS, D_MODEL), lambda i: (i, 0)),
            pl.BlockSpec((ROWS, D_MODEL), lambda i: (i, 0)),
            pl.BlockSpec((ROWS // BLOCK, D_MODEL), lambda i: (i, 0)),
        ],
        out_shape=[
            jax.ShapeDtypeStruct((L, D_MODEL), _BF),
            jax.ShapeDtypeStruct((L, D_MODEL), _BF),
            jax.ShapeDtypeStruct((L, D_MODEL), _BF),
            jax.ShapeDtypeStruct((N_BLOCKS, D_MODEL), jnp.float32),
        ],
    )(x2, wq16, wk16, wv16, cos, sin)

    starts = pl.pallas_call(
        _indexer_kernel,
        in_specs=[
            _full((N_BLOCKS, D_MODEL)),
            _full((D_MODEL, D_MODEL)),
            _full((D_MODEL, D_MODEL)),
            _full((D_MODEL, N_IDX_HEADS)),
        ],
        out_specs=_full((N_BLOCKS, N_SEL)),
        out_shape=jax.ShapeDtypeStruct((N_BLOCKS, N_SEL), jnp.int32),
    )(xb, idx_wq, idx_wk, idx_wg)

    out = pl.pallas_call(
        _attn_kernel,
        grid_spec=pltpu.PrefetchScalarGridSpec(
            num_scalar_prefetch=1,
            grid=(N_BLOCKS // QB_PER_STEP,),
            in_specs=[
                pl.BlockSpec((QB_PER_STEP * BLOCK, D_MODEL),
                             lambda i, sref: (i, 0)),
                pl.BlockSpec((L, D_MODEL), lambda i, sref: (0, 0)),
                pl.BlockSpec((L, D_MODEL), lambda i, sref: (0, 0)),
                pl.BlockSpec((D_MODEL, D_MODEL), lambda i, sref: (0, 0)),
            ],
            out_specs=pl.BlockSpec((QB_PER_STEP * BLOCK, D_MODEL),
                                   lambda i, sref: (i, 0)),
            scratch_shapes=[
                pltpu.VMEM((QB_PER_STEP * BLOCK, D_MODEL), _BF),
            ],
        ),
        out_shape=jax.ShapeDtypeStruct((L, D_MODEL), _BF),
    )(starts.reshape(-1), q_r, k_r, v, wo16)

    return out[None]


# speedup vs baseline: 1.1017x; 1.0212x over previous
import jax
import jax.numpy as jnp
from jax import lax
from jax.experimental import pallas as pl
from jax.experimental.pallas import tpu as pltpu

L = 2048
D_MODEL = 768
N_HEADS = 12
HEAD_DIM = 64
HALF = 32
BLOCK = 32
N_BLOCKS = L // BLOCK
N_SEL = 4
SEL_TOK = N_SEL * BLOCK
N_IDX_HEADS = 4
IDX_HD = D_MODEL // N_IDX_HEADS
ROWS = 512

_BF = jnp.bfloat16


def _proj_kernel(x_ref, wq_ref, wk_ref, wv_ref, cos_ref, sin_ref,
                 q_ref, k_ref, v_ref, xb_ref):
    x = x_ref[...]
    x16 = x.astype(_BF)
    cos = cos_ref[...]
    sin = sin_ref[...]
    lane = lax.broadcasted_iota(jnp.int32, (ROWS, D_MODEL), 1)
    first_half = (lane % HEAD_DIM) < HALF

    def rope_proj(w_ref, o_ref):
        t = lax.dot_general(x16, w_ref[...], (((1,), (0,)), ((), ())),
                            preferred_element_type=jnp.float32).astype(_BF)
        swap = jnp.where(first_half,
                         jnp.roll(t, -HALF, axis=1),
                         jnp.roll(t, HALF, axis=1))
        o_ref[...] = t * cos + swap * sin

    rope_proj(wq_ref, q_ref)
    rope_proj(wk_ref, k_ref)
    v = lax.dot_general(x16, wv_ref[...], (((1,), (0,)), ((), ())),
                        preferred_element_type=jnp.float32)
    v_ref[...] = v.astype(_BF)
    xb_ref[...] = jnp.mean(x.reshape(ROWS // BLOCK, BLOCK, D_MODEL), axis=1)


def _indexer_kernel(xb_ref, wqi_ref, wki_ref, wgi_ref, starts_ref):
    xb = xb_ref[...]
    q = lax.dot_general(xb, wqi_ref[...], (((1,), (0,)), ((), ())),
                        preferred_element_type=jnp.float32)
    k = lax.dot_general(xb, wki_ref[...], (((1,), (0,)), ((), ())),
                        preferred_element_type=jnp.float32)
    w = lax.dot_general(xb, wgi_ref[...], (((1,), (0,)), ((), ())),
                        preferred_element_type=jnp.float32)
    scores = jnp.zeros((N_BLOCKS, N_BLOCKS), jnp.float32)
    for h in range(N_IDX_HEADS):
        qh = q[:, h * IDX_HD:(h + 1) * IDX_HD]
        kh = k[:, h * IDX_HD:(h + 1) * IDX_HD]
        qk = lax.dot_general(qh, kh, (((1,), (1,)), ((), ())),
                             preferred_element_type=jnp.float32)
        scores = scores + jnp.maximum(qk, 0.0) * w[:, h:h + 1]
    iota = lax.broadcasted_iota(jnp.int32, (N_BLOCKS, N_BLOCKS), 1)
    cols = []
    for _ in range(N_SEL):
        m = jnp.max(scores, axis=1, keepdims=True)
        idx = jnp.min(jnp.where(scores == m, iota, N_BLOCKS), axis=1,
                      keepdims=True)
        cols.append(idx)
        scores = jnp.where(iota == idx, -jnp.inf, scores)
    starts_ref[...] = jnp.concatenate(cols, axis=1) * BLOCK


QB_PER_STEP = 16


def _attn_kernel(starts_ref, q_ref, k_ref, v_ref, wo_ref, o_ref, acc_ref):
    step = pl.program_id(0)
    for qq in range(QB_PER_STEP):
        qb = step * QB_PER_STEP + qq
        ksel = jnp.concatenate(
            [k_ref[pl.ds(pl.multiple_of(starts_ref[qb * N_SEL + s], BLOCK),
                         BLOCK), :] for s in range(N_SEL)], axis=0)
        vsel = jnp.concatenate(
            [v_ref[pl.ds(pl.multiple_of(starts_ref[qb * N_SEL + s], BLOCK),
                         BLOCK), :] for s in range(N_SEL)], axis=0)
        q = q_ref[qq * BLOCK:(qq + 1) * BLOCK, :]
        es = []
        for h in range(N_HEADS):
            sl = slice(h * HEAD_DIM, (h + 1) * HEAD_DIM)
            s32 = lax.dot_general(q[:, sl], ksel[:, sl],
                                  (((1,), (1,)), ((), ())),
                                  preferred_element_type=jnp.float32)
            sc = s32.astype(_BF).astype(jnp.float32) * 0.125
            es.append(jnp.exp(jnp.minimum(sc, 80.0)).astype(_BF))
        pvs, dens = [], []
        for h in range(N_HEADS):
            sl = slice(h * HEAD_DIM, (h + 1) * HEAD_DIM)
            dens.append(jnp.sum(es[h].astype(jnp.float32), axis=1,
                                keepdims=True))
            pvs.append(lax.dot_general(es[h], vsel[:, sl],
                                       (((1,), (0,)), ((), ())),
                                       preferred_element_type=jnp.float32))
        for h in range(N_HEADS):
            sl = slice(h * HEAD_DIM, (h + 1) * HEAD_DIM)
            acc_ref[qq * BLOCK:(qq + 1) * BLOCK, sl] = (
                pvs[h] / dens[h]).astype(_BF)
    o_ref[...] = lax.dot_general(acc_ref[...], wo_ref[...],
                                 (((1,), (0,)), ((), ())),
                                 preferred_element_type=jnp.float32).astype(_BF)


def _full(shape):
    nd = len(shape)
    return pl.BlockSpec(shape, lambda *_: (0,) * nd)


def kernel(x, wq, wk, wv, wo, idx_wq, idx_wk, idx_wg):
    x2 = x[0]
    wq16 = wq.astype(_BF)
    wk16 = wk.astype(_BF)
    wv16 = wv.astype(_BF)
    wo16 = wo.astype(_BF)

    inv = 1.0 / (10000.0 ** (jnp.arange(HALF, dtype=jnp.float32) / HALF))
    fr = jnp.outer(jnp.arange(L, dtype=jnp.float32), inv)
    c, s = jnp.cos(fr), jnp.sin(fr)
    cos = jnp.tile(jnp.concatenate([c, c], axis=1), (1, N_HEADS)).astype(_BF)
    sin = jnp.tile(jnp.concatenate([-s, s], axis=1), (1, N_HEADS)).astype(_BF)

    q_r, k_r, v, xb = pl.pallas_call(
        _proj_kernel,
        grid=(L // ROWS,),
        in_specs=[
            pl.BlockSpec((ROWS, D_MODEL), lambda i: (i, 0)),
            _full((D_MODEL, D_MODEL)),
            _full((D_MODEL, D_MODEL)),
            _full((D_MODEL, D_MODEL)),
            pl.BlockSpec((ROWS, D_MODEL), lambda i: (i, 0)),
            pl.BlockSpec((ROWS, D_MODEL), lambda i: (i, 0)),
        ],
        out_specs=[
            pl.BlockSpec((ROWS, D_MODEL), lambda i: (i, 0)),
            pl.BlockSpec((ROWS, D_MODEL), lambda i: (i, 0)),
            pl.BlockSpec((ROWS, D_MODEL), lambda i: (i, 0)),
            pl.BlockSpec((ROWS // BLOCK, D_MODEL), lambda i: (i, 0)),
        ],
        out_shape=[
            jax.ShapeDtypeStruct((L, D_MODEL), _BF),
            jax.ShapeDtypeStruct((L, D_MODEL), _BF),
            jax.ShapeDtypeStruct((L, D_MODEL), _BF),
            jax.ShapeDtypeStruct((N_BLOCKS, D_MODEL), jnp.float32),
        ],
    )(x2, wq16, wk16, wv16, cos, sin)

    starts = pl.pallas_call(
        _indexer_kernel,
        in_specs=[
            _full((N_BLOCKS, D_MODEL)),
            _full((D_MODEL, D_MODEL)),
            _full((D_MODEL, D_MODEL)),
            _full((D_MODEL, N_IDX_HEADS)),
        ],
        out_specs=_full((N_BLOCKS, N_SEL)),
        out_shape=jax.ShapeDtypeStruct((N_BLOCKS, N_SEL), jnp.int32),
    )(xb, idx_wq, idx_wk, idx_wg)

    out = pl.pallas_call(
        _attn_kernel,
        grid_spec=pltpu.PrefetchScalarGridSpec(
            num_scalar_prefetch=1,
            grid=(N_BLOCKS // QB_PER_STEP,),
            in_specs=[
                pl.BlockSpec((QB_PER_STEP * BLOCK, D_MODEL),
                             lambda i, sref: (i, 0)),
                pl.BlockSpec((L, D_MODEL), lambda i, sref: (0, 0)),
                pl.BlockSpec((L, D_MODEL), lambda i, sref: (0, 0)),
                pl.BlockSpec((D_MODEL, D_MODEL), lambda i, sref: (0, 0)),
            ],
            out_specs=pl.BlockSpec((QB_PER_STEP * BLOCK, D_MODEL),
                                   lambda i, sref: (i, 0)),
            scratch_shapes=[
                pltpu.VMEM((QB_PER_STEP * BLOCK, D_MODEL), _BF),
            ],
        ),
        out_shape=jax.ShapeDtypeStruct((L, D_MODEL), _BF),
    )(starts.reshape(-1), q_r, k_r, v, wo16)

    return out[None]
